# baseline (speedup 1.0000x reference)
import jax
import jax.numpy as jnp
from jax import lax
import jax.experimental.pallas as pl
from jax.experimental.pallas import tpu as pltpu
from jax.experimental.pallas import tpu_sc as plsc

N = 10000
E = 320000
D = 128

NC = 2
NS = 16

CHUNK = 64
EPT = E // NS
NCH = 320
EPT_PAD = NCH * CHUNK
SB = 16
NSB = NCH // SB
DUMP = N
NPAD = 10240
ZR = NPAD // NS
WR = 624
WREM = N - WR * NS

_MESH = dict(core_axis_name="c", subcore_axis_name="s", num_cores=NC,
             num_subcores=NS)


def _deg_body(dstb, zeros1, deg_out, idx_v, ones_v, acc):
    c = lax.axis_index("c")
    s = lax.axis_index("s")
    pltpu.sync_copy(zeros1, acc.at[pl.ds(s * ZR, ZR)])
    for k in range(CHUNK // 16):
        ones_v[pl.ds(k * 16, 16)] = jnp.ones((16,), jnp.float32)
    pltpu.sync_copy(dstb.at[c].at[s], idx_v)
    plsc.subcore_barrier()

    @pl.loop(0, NCH)
    def _(j):
        pltpu.sync_copy(ones_v, acc.at[idx_v.at[j]], add=True)

    plsc.subcore_barrier()
    pltpu.sync_copy(acc.at[pl.ds(s * ZR, ZR)], deg_out.at[c].at[pl.ds(s * ZR, ZR)])


NPAD_A = 10016
ZRA = NPAD_A // NS


def _compact(rows, cmp):
    @pl.loop(0, CHUNK)
    def _(k):
        for ccol in range(D // 16):
            cmp[k, pl.ds(ccol * 16, 16)] = rows[k, 0, pl.ds(ccol * 16, 16)]


def _agg_body(mpp, srcb, dstb, zeros2, agg_out, sidx, didx, rows0, rows1,
              cmp, acc, gsem0, gsem1, ssem0, ssem1):
    c = lax.axis_index("c")
    s = lax.axis_index("s")
    pltpu.sync_copy(zeros2, acc.at[pl.ds(s * ZRA, ZRA)])
    mppc = mpp.at[c]
    my_src = srcb.at[c].at[s]
    my_dst = dstb.at[c].at[s]
    plsc.subcore_barrier()

    def swait(k):
        pltpu.make_async_copy(cmp, acc.at[didx.at[k]], ssem0).wait()

    def sgo(k):
        pltpu.async_copy(cmp, acc.at[didx.at[k]], ssem0, add=True)

    @pl.loop(0, NSB)
    def _(t):
        pltpu.sync_copy(my_src.at[pl.ds(t * SB, SB)], sidx)
        pltpu.sync_copy(my_dst.at[pl.ds(t * SB, SB)], didx)
        pltpu.async_copy(mppc.at[sidx.at[0]], rows0, gsem0)
        pltpu.async_copy(mppc.at[sidx.at[1]], rows1, gsem1)

        pltpu.make_async_copy(mppc.at[sidx.at[0]], rows0, gsem0).wait()
        _compact(rows0, cmp)
        pltpu.async_copy(mppc.at[sidx.at[2]], rows0, gsem0)
        sgo(0)
        pltpu.make_async_copy(mppc.at[sidx.at[1]], rows1, gsem1).wait()
        swait(0)
        _compact(rows1, cmp)
        pltpu.async_copy(mppc.at[sidx.at[3]], rows1, gsem1)
        sgo(1)

        @pl.loop(2, SB - 2, step=2)
        def _(j):
            pltpu.make_async_copy(mppc.at[sidx.at[j]], rows0, gsem0).wait()
            swait(j - 1)
            _compact(rows0, cmp)
            pltpu.async_copy(mppc.at[sidx.at[j + 2]], rows0, gsem0)
            sgo(j)
            pltpu.make_async_copy(mppc.at[sidx.at[j + 1]], rows1, gsem1).wait()
            swait(j)
            _compact(rows1, cmp)
            pltpu.async_copy(mppc.at[sidx.at[j + 3]], rows1, gsem1)
            sgo(j + 1)

        pltpu.make_async_copy(mppc.at[sidx.at[SB - 2]], rows0, gsem0).wait()
        swait(SB - 3)
        _compact(rows0, cmp)
        sgo(SB - 2)
        pltpu.make_async_copy(mppc.at[sidx.at[SB - 1]], rows1, gsem1).wait()
        swait(SB - 2)
        _compact(rows1, cmp)
        sgo(SB - 1)
        swait(SB - 1)

    plsc.subcore_barrier()
    pltpu.sync_copy(acc.at[pl.ds(s * WR, WR)], agg_out.at[c].at[pl.ds(s * WR, WR)])

    @pl.when(s == NS - 1)
    def _():
        pltpu.sync_copy(acc.at[pl.ds(WR * NS, WREM)],
                        agg_out.at[c].at[pl.ds(WR * NS, WREM)])


_deg_kernel = pl.kernel(
    _deg_body,
    out_type=jax.ShapeDtypeStruct((NC, NPAD), jnp.float32),
    mesh=plsc.VectorSubcoreMesh(**_MESH),
    scratch_types=[
        pltpu.VMEM((NCH, CHUNK), jnp.int32),
        pltpu.VMEM((CHUNK,), jnp.float32),
        pltpu.VMEM_SHARED((NPAD,), jnp.float32),
    ],
)

_agg_kernel = pl.kernel(
    _agg_body,
    out_type=jax.ShapeDtypeStruct((NC, N, D), jnp.float32),
    mesh=plsc.VectorSubcoreMesh(**_MESH),
    scratch_types=[
        pltpu.VMEM((SB, CHUNK), jnp.int32),
        pltpu.VMEM((SB, CHUNK), jnp.int32),
        pltpu.VMEM((CHUNK, 2, D), jnp.float32),
        pltpu.VMEM((CHUNK, 2, D), jnp.float32),
        pltpu.VMEM((CHUNK, D), jnp.float32),
        pltpu.VMEM_SHARED((NPAD_A, D), jnp.float32),
        pltpu.SemaphoreType.DMA,
        pltpu.SemaphoreType.DMA,
        pltpu.SemaphoreType.DMA,
        pltpu.SemaphoreType.DMA,
    ],
)


BLK = 1000
GRID = N // BLK


def _tc1_body(x_ref, w0_ref, w1_ref, dv_ref, mpp_ref):
    xb = x_ref[...]
    dv = dv_ref[...]
    p0 = dv[0] * jnp.dot(xb, w0_ref[...], preferred_element_type=jnp.float32)
    p1 = dv[1] * jnp.dot(xb, w1_ref[...], preferred_element_type=jnp.float32)
    mpp_ref[0, :, 0] = p0
    mpp_ref[0, :, 1] = p0
    mpp_ref[1, :, 0] = p1
    mpp_ref[1, :, 1] = p1


def _tc2_body(agg_ref, mpp_ref, dv_ref, b_ref, w0_ref, w1_ref, mpp2_ref):
    agg = agg_ref[...]
    mp0 = mpp_ref[0, :, 0]
    mp1 = mpp_ref[1, :, 0]
    dv = dv_ref[...]
    b = b_ref[...]
    h = dv[0] * (agg[0] + mp0) + b[0] + dv[1] * (agg[1] + mp1) + b[1]
    h = jnp.maximum(h, 0.0)
    p0 = dv[0] * jnp.dot(h, w0_ref[...], preferred_element_type=jnp.float32)
    p1 = dv[1] * jnp.dot(h, w1_ref[...], preferred_element_type=jnp.float32)
    mpp2_ref[0, :, 0] = p0
    mpp2_ref[0, :, 1] = p0
    mpp2_ref[1, :, 0] = p1
    mpp2_ref[1, :, 1] = p1


def _tc3_body(agg_ref, mpp_ref, dv_ref, b_ref, out_ref):
    agg = agg_ref[...]
    mp0 = mpp_ref[0, :, 0]
    mp1 = mpp_ref[1, :, 0]
    dv = dv_ref[...]
    b = b_ref[...]
    out_ref[...] = (dv[0] * (agg[0] + mp0) + b[0]
                    + dv[1] * (agg[1] + mp1) + b[1])


_spec_x = pl.BlockSpec((BLK, D), lambda i: (i, 0))
_spec_w = pl.BlockSpec((D, D), lambda i: (0, 0))
_spec_dv = pl.BlockSpec((NC, BLK, 1), lambda i: (0, i, 0))
_spec_mp = pl.BlockSpec((NC, BLK, D), lambda i: (0, i, 0))
_spec_mpp = pl.BlockSpec((NC, BLK, 2, D), lambda i: (0, i, 0, 0))
_spec_b = pl.BlockSpec((NC, 1, D), lambda i: (0, 0, 0))
_spec_out = pl.BlockSpec((BLK, D), lambda i: (i, 0))

_tc1 = pl.pallas_call(
    _tc1_body,
    grid=(GRID,),
    in_specs=[_spec_x, _spec_w, _spec_w, _spec_dv],
    out_specs=_spec_mpp,
    out_shape=jax.ShapeDtypeStruct((NC, N, 2, D), jnp.float32),
)

_tc2 = pl.pallas_call(
    _tc2_body,
    grid=(GRID,),
    in_specs=[_spec_mp, _spec_mpp, _spec_dv, _spec_b, _spec_w, _spec_w],
    out_specs=_spec_mpp,
    out_shape=jax.ShapeDtypeStruct((NC, N, 2, D), jnp.float32),
)

_tc3 = pl.pallas_call(
    _tc3_body,
    grid=(GRID,),
    in_specs=[_spec_mp, _spec_mpp, _spec_dv, _spec_b],
    out_specs=_spec_out,
    out_shape=jax.ShapeDtypeStruct((N, D), jnp.float32),
)


def _prep_edges(ei):
    src = ei[0].reshape(NS, EPT)
    dst = ei[1].reshape(NS, EPT)
    pad = EPT_PAD - EPT
    src = jnp.pad(src, ((0, 0), (0, pad)))
    dst = jnp.pad(dst, ((0, 0), (0, pad)), constant_values=DUMP)
    return src.reshape(NS, NCH, CHUNK), dst.reshape(NS, NCH, CHUNK)


def kernel(x, edge_index_0, edge_index_1, W1_0, b1_0, W1_1, b1_1,
           W2_0, b2_0, W2_1, b2_1):
    s0, d0 = _prep_edges(edge_index_0)
    s1, d1 = _prep_edges(edge_index_1)
    srcb = jnp.stack([s0, s1])
    dstb = jnp.stack([d0, d1])
    zeros1 = jnp.zeros((ZR,), jnp.float32)
    zeros2 = jnp.zeros((ZRA, D), jnp.float32)

    degc = _deg_kernel(dstb, zeros1)
    dinv = lax.rsqrt(degc[:, :N] + 1.0)
    dv = dinv[:, :, None]

    b1s = jnp.stack([b1_0, b1_1]).reshape(NC, 1, D)
    b2s = jnp.stack([b2_0, b2_1]).reshape(NC, 1, D)

    mpp1 = _tc1(x, W1_0, W1_1, dv)
    agg1 = _agg_kernel(mpp1, srcb, dstb, zeros2)
    mpp2 = _tc2(agg1, mpp1, dv, b1s, W2_0, W2_1)
    agg2 = _agg_kernel(mpp2, srcb, dstb, zeros2)
    return _tc3(agg2, mpp2, dv, b2s)


# speedup vs baseline: 1.0131x; 1.0131x over previous
import jax
import jax.numpy as jnp
from jax import lax
import jax.experimental.pallas as pl
from jax.experimental.pallas import tpu as pltpu
from jax.experimental.pallas import tpu_sc as plsc

N = 10000
E = 320000
D = 128

NC = 2
NS = 16

CHUNK = 64
EPT = E // NS
NCH = 320
EPT_PAD = NCH * CHUNK
SB = 16
NSB = NCH // SB
DUMP = N
NPAD = 10240
ZR = NPAD // NS
WR = 624
WREM = N - WR * NS

_MESH = dict(core_axis_name="c", subcore_axis_name="s", num_cores=NC,
             num_subcores=NS)


def _deg_body(dstb, zeros1, deg_out, idx_v, ones_v, acc):
    c = lax.axis_index("c")
    s = lax.axis_index("s")
    pltpu.sync_copy(zeros1, acc.at[pl.ds(s * ZR, ZR)])
    for k in range(CHUNK // 16):
        ones_v[pl.ds(k * 16, 16)] = jnp.ones((16,), jnp.float32)
    pltpu.sync_copy(dstb.at[c].at[s], idx_v)
    plsc.subcore_barrier()

    @pl.loop(0, NCH)
    def _(j):
        pltpu.sync_copy(ones_v, acc.at[idx_v.at[j]], add=True)

    plsc.subcore_barrier()
    pltpu.sync_copy(acc.at[pl.ds(s * ZR, ZR)], deg_out.at[c].at[pl.ds(s * ZR, ZR)])


NPAD_A = 10016
ZRA = NPAD_A // NS


def _compact(rows, cmp):
    @pl.loop(0, CHUNK, unroll=8)
    def _(k):
        for ccol in range(D // 16):
            cmp[k, pl.ds(ccol * 16, 16)] = rows[k, 0, pl.ds(ccol * 16, 16)]


def _agg_body(mpp, srcb, dstb, zeros2, agg_out, sidx, didx, rows0, rows1,
              cmp, acc, gsem0, gsem1, ssem0, ssem1):
    c = lax.axis_index("c")
    s = lax.axis_index("s")
    pltpu.sync_copy(zeros2, acc.at[pl.ds(s * ZRA, ZRA)])
    mppc = mpp.at[c]
    my_src = srcb.at[c].at[s]
    my_dst = dstb.at[c].at[s]
    plsc.subcore_barrier()

    def swait(k):
        pltpu.make_async_copy(cmp, acc.at[didx.at[k]], ssem0).wait()

    def sgo(k):
        pltpu.async_copy(cmp, acc.at[didx.at[k]], ssem0, add=True)

    @pl.loop(0, NSB)
    def _(t):
        pltpu.sync_copy(my_src.at[pl.ds(t * SB, SB)], sidx)
        pltpu.sync_copy(my_dst.at[pl.ds(t * SB, SB)], didx)
        pltpu.async_copy(mppc.at[sidx.at[0]], rows0, gsem0)
        pltpu.async_copy(mppc.at[sidx.at[1]], rows1, gsem1)

        pltpu.make_async_copy(mppc.at[sidx.at[0]], rows0, gsem0).wait()
        _compact(rows0, cmp)
        pltpu.async_copy(mppc.at[sidx.at[2]], rows0, gsem0)
        sgo(0)
        pltpu.make_async_copy(mppc.at[sidx.at[1]], rows1, gsem1).wait()
        swait(0)
        _compact(rows1, cmp)
        pltpu.async_copy(mppc.at[sidx.at[3]], rows1, gsem1)
        sgo(1)

        @pl.loop(2, SB - 2, step=2)
        def _(j):
            pltpu.make_async_copy(mppc.at[sidx.at[j]], rows0, gsem0).wait()
            swait(j - 1)
            _compact(rows0, cmp)
            pltpu.async_copy(mppc.at[sidx.at[j + 2]], rows0, gsem0)
            sgo(j)
            pltpu.make_async_copy(mppc.at[sidx.at[j + 1]], rows1, gsem1).wait()
            swait(j)
            _compact(rows1, cmp)
            pltpu.async_copy(mppc.at[sidx.at[j + 3]], rows1, gsem1)
            sgo(j + 1)

        pltpu.make_async_copy(mppc.at[sidx.at[SB - 2]], rows0, gsem0).wait()
        swait(SB - 3)
        _compact(rows0, cmp)
        sgo(SB - 2)
        pltpu.make_async_copy(mppc.at[sidx.at[SB - 1]], rows1, gsem1).wait()
        swait(SB - 2)
        _compact(rows1, cmp)
        sgo(SB - 1)
        swait(SB - 1)

    plsc.subcore_barrier()
    pltpu.sync_copy(acc.at[pl.ds(s * WR, WR)], agg_out.at[c].at[pl.ds(s * WR, WR)])

    @pl.when(s == NS - 1)
    def _():
        pltpu.sync_copy(acc.at[pl.ds(WR * NS, WREM)],
                        agg_out.at[c].at[pl.ds(WR * NS, WREM)])


_deg_kernel = pl.kernel(
    _deg_body,
    out_type=jax.ShapeDtypeStruct((NC, NPAD), jnp.float32),
    mesh=plsc.VectorSubcoreMesh(**_MESH),
    scratch_types=[
        pltpu.VMEM((NCH, CHUNK), jnp.int32),
        pltpu.VMEM((CHUNK,), jnp.float32),
        pltpu.VMEM_SHARED((NPAD,), jnp.float32),
    ],
)

_agg_kernel = pl.kernel(
    _agg_body,
    out_type=jax.ShapeDtypeStruct((NC, N, D), jnp.float32),
    mesh=plsc.VectorSubcoreMesh(**_MESH),
    scratch_types=[
        pltpu.VMEM((SB, CHUNK), jnp.int32),
        pltpu.VMEM((SB, CHUNK), jnp.int32),
        pltpu.VMEM((CHUNK, 2, D), jnp.float32),
        pltpu.VMEM((CHUNK, 2, D), jnp.float32),
        pltpu.VMEM((CHUNK, D), jnp.float32),
        pltpu.VMEM_SHARED((NPAD_A, D), jnp.float32),
        pltpu.SemaphoreType.DMA,
        pltpu.SemaphoreType.DMA,
        pltpu.SemaphoreType.DMA,
        pltpu.SemaphoreType.DMA,
    ],
)


BLK = 1000
GRID = N // BLK


def _tc1_body(x_ref, w0_ref, w1_ref, dv_ref, mpp_ref):
    xb = x_ref[...]
    dv = dv_ref[...]
    p0 = dv[0] * jnp.dot(xb, w0_ref[...], preferred_element_type=jnp.float32)
    p1 = dv[1] * jnp.dot(xb, w1_ref[...], preferred_element_type=jnp.float32)
    mpp_ref[0, :, 0] = p0
    mpp_ref[0, :, 1] = p0
    mpp_ref[1, :, 0] = p1
    mpp_ref[1, :, 1] = p1


def _tc2_body(agg_ref, mpp_ref, dv_ref, b_ref, w0_ref, w1_ref, mpp2_ref):
    agg = agg_ref[...]
    mp0 = mpp_ref[0, :, 0]
    mp1 = mpp_ref[1, :, 0]
    dv = dv_ref[...]
    b = b_ref[...]
    h = dv[0] * (agg[0] + mp0) + b[0] + dv[1] * (agg[1] + mp1) + b[1]
    h = jnp.maximum(h, 0.0)
    p0 = dv[0] * jnp.dot(h, w0_ref[...], preferred_element_type=jnp.float32)
    p1 = dv[1] * jnp.dot(h, w1_ref[...], preferred_element_type=jnp.float32)
    mpp2_ref[0, :, 0] = p0
    mpp2_ref[0, :, 1] = p0
    mpp2_ref[1, :, 0] = p1
    mpp2_ref[1, :, 1] = p1


def _tc3_body(agg_ref, mpp_ref, dv_ref, b_ref, out_ref):
    agg = agg_ref[...]
    mp0 = mpp_ref[0, :, 0]
    mp1 = mpp_ref[1, :, 0]
    dv = dv_ref[...]
    b = b_ref[...]
    out_ref[...] = (dv[0] * (agg[0] + mp0) + b[0]
                    + dv[1] * (agg[1] + mp1) + b[1])


_spec_x = pl.BlockSpec((BLK, D), lambda i: (i, 0))
_spec_w = pl.BlockSpec((D, D), lambda i: (0, 0))
_spec_dv = pl.BlockSpec((NC, BLK, 1), lambda i: (0, i, 0))
_spec_mp = pl.BlockSpec((NC, BLK, D), lambda i: (0, i, 0))
_spec_mpp = pl.BlockSpec((NC, BLK, 2, D), lambda i: (0, i, 0, 0))
_spec_b = pl.BlockSpec((NC, 1, D), lambda i: (0, 0, 0))
_spec_out = pl.BlockSpec((BLK, D), lambda i: (i, 0))

_tc1 = pl.pallas_call(
    _tc1_body,
    grid=(GRID,),
    in_specs=[_spec_x, _spec_w, _spec_w, _spec_dv],
    out_specs=_spec_mpp,
    out_shape=jax.ShapeDtypeStruct((NC, N, 2, D), jnp.float32),
)

_tc2 = pl.pallas_call(
    _tc2_body,
    grid=(GRID,),
    in_specs=[_spec_mp, _spec_mpp, _spec_dv, _spec_b, _spec_w, _spec_w],
    out_specs=_spec_mpp,
    out_shape=jax.ShapeDtypeStruct((NC, N, 2, D), jnp.float32),
)

_tc3 = pl.pallas_call(
    _tc3_body,
    grid=(GRID,),
    in_specs=[_spec_mp, _spec_mpp, _spec_dv, _spec_b],
    out_specs=_spec_out,
    out_shape=jax.ShapeDtypeStruct((N, D), jnp.float32),
)


def _prep_edges(ei):
    src = ei[0].reshape(NS, EPT)
    dst = ei[1].reshape(NS, EPT)
    pad = EPT_PAD - EPT
    src = jnp.pad(src, ((0, 0), (0, pad)))
    dst = jnp.pad(dst, ((0, 0), (0, pad)), constant_values=DUMP)
    return src.reshape(NS, NCH, CHUNK), dst.reshape(NS, NCH, CHUNK)


def kernel(x, edge_index_0, edge_index_1, W1_0, b1_0, W1_1, b1_1,
           W2_0, b2_0, W2_1, b2_1):
    s0, d0 = _prep_edges(edge_index_0)
    s1, d1 = _prep_edges(edge_index_1)
    srcb = jnp.stack([s0, s1])
    dstb = jnp.stack([d0, d1])
    zeros1 = jnp.zeros((ZR,), jnp.float32)
    zeros2 = jnp.zeros((ZRA, D), jnp.float32)

    degc = _deg_kernel(dstb, zeros1)
    dinv = lax.rsqrt(degc[:, :N] + 1.0)
    dv = dinv[:, :, None]

    b1s = jnp.stack([b1_0, b1_1]).reshape(NC, 1, D)
    b2s = jnp.stack([b2_0, b2_1]).reshape(NC, 1, D)

    mpp1 = _tc1(x, W1_0, W1_1, dv)
    agg1 = _agg_kernel(mpp1, srcb, dstb, zeros2)
    mpp2 = _tc2(agg1, mpp1, dv, b1s, W2_0, W2_1)
    agg2 = _agg_kernel(mpp2, srcb, dstb, zeros2)
    return _tc3(agg2, mpp2, dv, b2s)


# speedup vs baseline: 1.8747x; 1.8504x over previous
import jax
import jax.numpy as jnp
from jax import lax
import jax.experimental.pallas as pl
from jax.experimental.pallas import tpu as pltpu
from jax.experimental.pallas import tpu_sc as plsc

N = 10000
E = 320000
D = 128

NC = 2
NS = 16

CHUNK = 64
EPT = E // NS
NCH = 320
EPT_PAD = NCH * CHUNK
SB = 64
NSB = NCH // SB
DUMP = N
NPAD = 10240
ZR = NPAD // NS
WR = 624
WREM = N - WR * NS

_MESH = dict(core_axis_name="c", subcore_axis_name="s", num_cores=NC,
             num_subcores=NS)


def _deg_body(dstb, zeros1, deg_out, idx_v, ones_v, acc):
    c = lax.axis_index("c")
    s = lax.axis_index("s")
    pltpu.sync_copy(zeros1, acc.at[pl.ds(s * ZR, ZR)])
    for k in range(CHUNK // 16):
        ones_v[pl.ds(k * 16, 16)] = jnp.ones((16,), jnp.float32)
    pltpu.sync_copy(dstb.at[c].at[s], idx_v)
    plsc.subcore_barrier()

    @pl.loop(0, NCH)
    def _(j):
        pltpu.sync_copy(ones_v, acc.at[idx_v.at[j]], add=True)

    plsc.subcore_barrier()
    pltpu.sync_copy(acc.at[pl.ds(s * ZR, ZR)], deg_out.at[c].at[pl.ds(s * ZR, ZR)])


def _agg_body(mpp, srcb, dstb, zeros2, agg_out, sidx, didx, r0, r1, r2, r3,
              acc, g0, g1, g2, g3, s0, s1, s2, s3):
    c = lax.axis_index("c")
    s = lax.axis_index("s")
    pltpu.sync_copy(zeros2, acc.at[pl.ds(s * ZR, ZR)])
    mpc = mpp.at[c]
    my_src = srcb.at[c].at[s]
    my_dst = dstb.at[c].at[s]
    plsc.subcore_barrier()
    rows = [r0, r1, r2, r3]
    gsem = [g0, g1, g2, g3]
    ssem = [s0, s1, s2, s3]

    def gwait(j, b):
        pltpu.make_async_copy(mpc.at[sidx.at[j]], rows[b], gsem[b]).wait()

    def ggo(j, b):
        pltpu.async_copy(mpc.at[sidx.at[j]], rows[b], gsem[b])

    def swait(j, b):
        pltpu.make_async_copy(rows[b], acc.at[didx.at[j]], ssem[b]).wait()

    def sgo(j, b):
        pltpu.async_copy(rows[b], acc.at[didx.at[j]], ssem[b], add=True)

    @pl.loop(0, NSB)
    def _(t):
        pltpu.sync_copy(my_src.at[pl.ds(t * SB, SB)], sidx)
        pltpu.sync_copy(my_dst.at[pl.ds(t * SB, SB)], didx)
        for b in range(4):
            ggo(b, b)
        gwait(0, 0); sgo(0, 0)
        gwait(1, 1); sgo(1, 1)
        gwait(2, 2); sgo(2, 2); swait(0, 0); ggo(4, 0)
        gwait(3, 3); sgo(3, 3); swait(1, 1); ggo(5, 1)

        @pl.loop(4, SB - 4, step=4)
        def _(j):
            for b in range(4):
                jj = j + b
                gwait(jj, b)
                sgo(jj, b)
                swait(jj - 2, (b + 2) % 4)
                ggo(jj + 2, (b + 2) % 4)

        gwait(SB - 4, 0); sgo(SB - 4, 0); swait(SB - 6, 2); ggo(SB - 2, 2)
        gwait(SB - 3, 1); sgo(SB - 3, 1); swait(SB - 5, 3); ggo(SB - 1, 3)
        gwait(SB - 2, 2); sgo(SB - 2, 2); swait(SB - 4, 0)
        gwait(SB - 1, 3); sgo(SB - 1, 3); swait(SB - 3, 1)
        swait(SB - 2, 2)
        swait(SB - 1, 3)

    plsc.subcore_barrier()
    pltpu.sync_copy(acc.at[pl.ds(s * WR, WR)], agg_out.at[c].at[pl.ds(s * WR, WR)])

    @pl.when(s == NS - 1)
    def _():
        pltpu.sync_copy(acc.at[pl.ds(WR * NS, WREM)],
                        agg_out.at[c].at[pl.ds(WR * NS, WREM)])


_deg_kernel = pl.kernel(
    _deg_body,
    out_type=jax.ShapeDtypeStruct((NC, NPAD), jnp.float32),
    mesh=plsc.VectorSubcoreMesh(**_MESH),
    scratch_types=[
        pltpu.VMEM((NCH, CHUNK), jnp.int32),
        pltpu.VMEM((CHUNK,), jnp.float32),
        pltpu.VMEM_SHARED((NPAD,), jnp.float32),
    ],
)

_agg_kernel = pl.kernel(
    _agg_body,
    out_type=jax.ShapeDtypeStruct((NC, N, D), jnp.float32),
    mesh=plsc.VectorSubcoreMesh(**_MESH),
    scratch_types=[
        pltpu.VMEM((SB, CHUNK), jnp.int32),
        pltpu.VMEM((SB, CHUNK), jnp.int32),
        pltpu.VMEM((CHUNK, D), jnp.float32),
        pltpu.VMEM((CHUNK, D), jnp.float32),
        pltpu.VMEM((CHUNK, D), jnp.float32),
        pltpu.VMEM((CHUNK, D), jnp.float32),
        pltpu.VMEM_SHARED((NPAD, D), jnp.float32),
    ] + [pltpu.SemaphoreType.DMA] * 8,
)


BLK = 1000
GRID = N // BLK


def _tc1_body(x_ref, w0_ref, w1_ref, dv_ref, mp_ref):
    xb = x_ref[...]
    dv = dv_ref[...]
    mp_ref[0] = dv[0] * jnp.dot(xb, w0_ref[...], preferred_element_type=jnp.float32)
    mp_ref[1] = dv[1] * jnp.dot(xb, w1_ref[...], preferred_element_type=jnp.float32)


def _tc2_body(agg_ref, mp_ref, dv_ref, b_ref, w0_ref, w1_ref, mp2_ref):
    agg = agg_ref[...]
    mp = mp_ref[...]
    dv = dv_ref[...]
    b = b_ref[...]
    h = dv[0] * (agg[0] + mp[0]) + b[0] + dv[1] * (agg[1] + mp[1]) + b[1]
    h = jnp.maximum(h, 0.0)
    mp2_ref[0] = dv[0] * jnp.dot(h, w0_ref[...], preferred_element_type=jnp.float32)
    mp2_ref[1] = dv[1] * jnp.dot(h, w1_ref[...], preferred_element_type=jnp.float32)


def _tc3_body(agg_ref, mp_ref, dv_ref, b_ref, out_ref):
    agg = agg_ref[...]
    mp = mp_ref[...]
    dv = dv_ref[...]
    b = b_ref[...]
    out_ref[...] = (dv[0] * (agg[0] + mp[0]) + b[0]
                    + dv[1] * (agg[1] + mp[1]) + b[1])


_spec_x = pl.BlockSpec((BLK, D), lambda i: (i, 0))
_spec_w = pl.BlockSpec((D, D), lambda i: (0, 0))
_spec_dv = pl.BlockSpec((NC, BLK, 1), lambda i: (0, i, 0))
_spec_mp = pl.BlockSpec((NC, BLK, D), lambda i: (0, i, 0))
_spec_b = pl.BlockSpec((NC, 1, D), lambda i: (0, 0, 0))
_spec_out = pl.BlockSpec((BLK, D), lambda i: (i, 0))

_tc1 = pl.pallas_call(
    _tc1_body,
    grid=(GRID,),
    in_specs=[_spec_x, _spec_w, _spec_w, _spec_dv],
    out_specs=_spec_mp,
    out_shape=jax.ShapeDtypeStruct((NC, N, D), jnp.float32),
)

_tc2 = pl.pallas_call(
    _tc2_body,
    grid=(GRID,),
    in_specs=[_spec_mp, _spec_mp, _spec_dv, _spec_b, _spec_w, _spec_w],
    out_specs=_spec_mp,
    out_shape=jax.ShapeDtypeStruct((NC, N, D), jnp.float32),
)

_tc3 = pl.pallas_call(
    _tc3_body,
    grid=(GRID,),
    in_specs=[_spec_mp, _spec_mp, _spec_dv, _spec_b],
    out_specs=_spec_out,
    out_shape=jax.ShapeDtypeStruct((N, D), jnp.float32),
)


def _prep_edges(ei):
    src = ei[0].reshape(NS, EPT)
    dst = ei[1].reshape(NS, EPT)
    pad = EPT_PAD - EPT
    src = jnp.pad(src, ((0, 0), (0, pad)))
    dst = jnp.pad(dst, ((0, 0), (0, pad)), constant_values=DUMP)
    return src.reshape(NS, NCH, CHUNK), dst.reshape(NS, NCH, CHUNK)


def kernel(x, edge_index_0, edge_index_1, W1_0, b1_0, W1_1, b1_1,
           W2_0, b2_0, W2_1, b2_1):
    s0, d0 = _prep_edges(edge_index_0)
    s1, d1 = _prep_edges(edge_index_1)
    srcb = jnp.stack([s0, s1])
    dstb = jnp.stack([d0, d1])
    zeros1 = jnp.zeros((ZR,), jnp.float32)
    zeros2 = jnp.zeros((ZR, D), jnp.float32)

    degc = _deg_kernel(dstb, zeros1)
    dinv = lax.rsqrt(degc[:, :N] + 1.0)
    dv = dinv[:, :, None]

    b1s = jnp.stack([b1_0, b1_1]).reshape(NC, 1, D)
    b2s = jnp.stack([b2_0, b2_1]).reshape(NC, 1, D)

    mp1 = _tc1(x, W1_0, W1_1, dv)
    agg1 = _agg_kernel(mp1, srcb, dstb, zeros2)
    mp2 = _tc2(agg1, mp1, dv, b1s, W2_0, W2_1)
    agg2 = _agg_kernel(mp2, srcb, dstb, zeros2)
    return _tc3(agg2, mp2, dv, b2s)


# speedup vs baseline: 1.8881x; 1.0071x over previous
import jax
import jax.numpy as jnp
from jax import lax
import jax.experimental.pallas as pl
from jax.experimental.pallas import tpu as pltpu
from jax.experimental.pallas import tpu_sc as plsc

N = 10000
E = 320000
D = 128

NC = 2
NS = 16

CHUNK = 64
EPT = E // NS
NCH = 320
EPT_PAD = NCH * CHUNK
SB = 64
NSB = NCH // SB
DUMP = N
NPAD = 10240
ZR = NPAD // NS
WR = 624
WREM = N - WR * NS

_MESH = dict(core_axis_name="c", subcore_axis_name="s", num_cores=NC,
             num_subcores=NS)


def _deg_body(dstb, zeros1, deg_out, idx_v, ones_v, acc, dsem0, dsem1):
    c = lax.axis_index("c")
    s = lax.axis_index("s")
    pltpu.sync_copy(zeros1, acc.at[pl.ds(s * ZR, ZR)])
    for k in range(CHUNK // 16):
        ones_v[pl.ds(k * 16, 16)] = jnp.ones((16,), jnp.float32)
    pltpu.sync_copy(dstb.at[c].at[s], idx_v)
    plsc.subcore_barrier()

    @pl.loop(0, NCH, step=2)
    def _(j):
        cp0 = pltpu.async_copy(ones_v, acc.at[idx_v.at[j]], dsem0, add=True)
        cp1 = pltpu.async_copy(ones_v, acc.at[idx_v.at[j + 1]], dsem1, add=True)
        cp0.wait()
        cp1.wait()

    plsc.subcore_barrier()
    pltpu.sync_copy(acc.at[pl.ds(s * ZR, ZR)], deg_out.at[c].at[pl.ds(s * ZR, ZR)])


def _agg_body(mpp, srcb, dstb, zeros2, agg_out, sidx, didx, r0, r1, r2, r3,
              acc, g0, g1, g2, g3, s0, s1, s2, s3):
    c = lax.axis_index("c")
    s = lax.axis_index("s")
    pltpu.sync_copy(zeros2, acc.at[pl.ds(s * ZR, ZR)])
    mpc = mpp.at[c]
    my_src = srcb.at[c].at[s]
    my_dst = dstb.at[c].at[s]
    plsc.subcore_barrier()
    rows = [r0, r1, r2, r3]
    gsem = [g0, g1, g2, g3]
    ssem = [s0, s1, s2, s3]

    def gwait(j, b):
        pltpu.make_async_copy(mpc.at[sidx.at[j]], rows[b], gsem[b]).wait()

    def ggo(j, b):
        pltpu.async_copy(mpc.at[sidx.at[j]], rows[b], gsem[b])

    def swait(j, b):
        pltpu.make_async_copy(rows[b], acc.at[didx.at[j]], ssem[b]).wait()

    def sgo(j, b):
        pltpu.async_copy(rows[b], acc.at[didx.at[j]], ssem[b], add=True)

    @pl.loop(0, NSB)
    def _(t):
        pltpu.sync_copy(my_src.at[pl.ds(t * SB, SB)], sidx)
        pltpu.sync_copy(my_dst.at[pl.ds(t * SB, SB)], didx)
        for b in range(4):
            ggo(b, b)
        gwait(0, 0); sgo(0, 0)
        gwait(1, 1); sgo(1, 1)
        gwait(2, 2); sgo(2, 2); swait(0, 0); ggo(4, 0)
        gwait(3, 3); sgo(3, 3); swait(1, 1); ggo(5, 1)

        @pl.loop(4, SB - 4, step=4)
        def _(j):
            for b in range(4):
                jj = j + b
                gwait(jj, b)
                sgo(jj, b)
                swait(jj - 2, (b + 2) % 4)
                ggo(jj + 2, (b + 2) % 4)

        gwait(SB - 4, 0); sgo(SB - 4, 0); swait(SB - 6, 2); ggo(SB - 2, 2)
        gwait(SB - 3, 1); sgo(SB - 3, 1); swait(SB - 5, 3); ggo(SB - 1, 3)
        gwait(SB - 2, 2); sgo(SB - 2, 2); swait(SB - 4, 0)
        gwait(SB - 1, 3); sgo(SB - 1, 3); swait(SB - 3, 1)
        swait(SB - 2, 2)
        swait(SB - 1, 3)

    plsc.subcore_barrier()
    pltpu.sync_copy(acc.at[pl.ds(s * WR, WR)], agg_out.at[c].at[pl.ds(s * WR, WR)])

    @pl.when(s == NS - 1)
    def _():
        pltpu.sync_copy(acc.at[pl.ds(WR * NS, WREM)],
                        agg_out.at[c].at[pl.ds(WR * NS, WREM)])


_deg_kernel = pl.kernel(
    _deg_body,
    out_type=jax.ShapeDtypeStruct((NC, NPAD), jnp.float32),
    mesh=plsc.VectorSubcoreMesh(**_MESH),
    scratch_types=[
        pltpu.VMEM((NCH, CHUNK), jnp.int32),
        pltpu.VMEM((CHUNK,), jnp.float32),
        pltpu.VMEM_SHARED((NPAD,), jnp.float32),
        pltpu.SemaphoreType.DMA,
        pltpu.SemaphoreType.DMA,
    ],
)

_agg_kernel = pl.kernel(
    _agg_body,
    out_type=jax.ShapeDtypeStruct((NC, N, D), jnp.float32),
    mesh=plsc.VectorSubcoreMesh(**_MESH),
    scratch_types=[
        pltpu.VMEM((SB, CHUNK), jnp.int32),
        pltpu.VMEM((SB, CHUNK), jnp.int32),
        pltpu.VMEM((CHUNK, D), jnp.float32),
        pltpu.VMEM((CHUNK, D), jnp.float32),
        pltpu.VMEM((CHUNK, D), jnp.float32),
        pltpu.VMEM((CHUNK, D), jnp.float32),
        pltpu.VMEM_SHARED((NPAD, D), jnp.float32),
    ] + [pltpu.SemaphoreType.DMA] * 8,
)


BLK = 1000
GRID = N // BLK


def _tc1_body(x_ref, w0_ref, w1_ref, dv_ref, mp_ref):
    xb = x_ref[...]
    dv = dv_ref[...]
    mp_ref[0] = dv[0] * jnp.dot(xb, w0_ref[...], preferred_element_type=jnp.float32)
    mp_ref[1] = dv[1] * jnp.dot(xb, w1_ref[...], preferred_element_type=jnp.float32)


def _tc2_body(agg_ref, mp_ref, dv_ref, b_ref, w0_ref, w1_ref, mp2_ref):
    agg = agg_ref[...]
    mp = mp_ref[...]
    dv = dv_ref[...]
    b = b_ref[...]
    h = dv[0] * (agg[0] + mp[0]) + b[0] + dv[1] * (agg[1] + mp[1]) + b[1]
    h = jnp.maximum(h, 0.0)
    mp2_ref[0] = dv[0] * jnp.dot(h, w0_ref[...], preferred_element_type=jnp.float32)
    mp2_ref[1] = dv[1] * jnp.dot(h, w1_ref[...], preferred_element_type=jnp.float32)


def _tc3_body(agg_ref, mp_ref, dv_ref, b_ref, out_ref):
    agg = agg_ref[...]
    mp = mp_ref[...]
    dv = dv_ref[...]
    b = b_ref[...]
    out_ref[...] = (dv[0] * (agg[0] + mp[0]) + b[0]
                    + dv[1] * (agg[1] + mp[1]) + b[1])


_spec_x = pl.BlockSpec((BLK, D), lambda i: (i, 0))
_spec_w = pl.BlockSpec((D, D), lambda i: (0, 0))
_spec_dv = pl.BlockSpec((NC, BLK, 1), lambda i: (0, i, 0))
_spec_mp = pl.BlockSpec((NC, BLK, D), lambda i: (0, i, 0))
_spec_b = pl.BlockSpec((NC, 1, D), lambda i: (0, 0, 0))
_spec_out = pl.BlockSpec((BLK, D), lambda i: (i, 0))

_tc1 = pl.pallas_call(
    _tc1_body,
    grid=(GRID,),
    in_specs=[_spec_x, _spec_w, _spec_w, _spec_dv],
    out_specs=_spec_mp,
    out_shape=jax.ShapeDtypeStruct((NC, N, D), jnp.float32),
)

_tc2 = pl.pallas_call(
    _tc2_body,
    grid=(GRID,),
    in_specs=[_spec_mp, _spec_mp, _spec_dv, _spec_b, _spec_w, _spec_w],
    out_specs=_spec_mp,
    out_shape=jax.ShapeDtypeStruct((NC, N, D), jnp.float32),
)

_tc3 = pl.pallas_call(
    _tc3_body,
    grid=(GRID,),
    in_specs=[_spec_mp, _spec_mp, _spec_dv, _spec_b],
    out_specs=_spec_out,
    out_shape=jax.ShapeDtypeStruct((N, D), jnp.float32),
)


def _prep_edges(ei):
    src = ei[0].reshape(NS, EPT)
    dst = ei[1].reshape(NS, EPT)
    pad = EPT_PAD - EPT
    src = jnp.pad(src, ((0, 0), (0, pad)))
    dst = jnp.pad(dst, ((0, 0), (0, pad)), constant_values=DUMP)
    return src.reshape(NS, NCH, CHUNK), dst.reshape(NS, NCH, CHUNK)


def kernel(x, edge_index_0, edge_index_1, W1_0, b1_0, W1_1, b1_1,
           W2_0, b2_0, W2_1, b2_1):
    s0, d0 = _prep_edges(edge_index_0)
    s1, d1 = _prep_edges(edge_index_1)
    srcb = jnp.stack([s0, s1])
    dstb = jnp.stack([d0, d1])
    zeros1 = jnp.zeros((ZR,), jnp.float32)
    zeros2 = jnp.zeros((ZR, D), jnp.float32)

    degc = _deg_kernel(dstb, zeros1)
    dinv = lax.rsqrt(degc[:, :N] + 1.0)
    dv = dinv[:, :, None]

    b1s = jnp.stack([b1_0, b1_1]).reshape(NC, 1, D)
    b2s = jnp.stack([b2_0, b2_1]).reshape(NC, 1, D)

    mp1 = _tc1(x, W1_0, W1_1, dv)
    agg1 = _agg_kernel(mp1, srcb, dstb, zeros2)
    mp2 = _tc2(agg1, mp1, dv, b1s, W2_0, W2_1)
    agg2 = _agg_kernel(mp2, srcb, dstb, zeros2)
    return _tc3(agg2, mp2, dv, b2s)


# speedup vs baseline: 1.9181x; 1.0159x over previous
import jax
import jax.numpy as jnp
from jax import lax
import jax.experimental.pallas as pl
from jax.experimental.pallas import tpu as pltpu
from jax.experimental.pallas import tpu_sc as plsc

N = 10000
E = 320000
D = 128

NC = 2
NS = 16

CHUNK = 80
EPT = E // NS
NCH = 256
EPT_PAD = NCH * CHUNK
SB = 32
NSB = NCH // SB
DUMP = N
NPAD = 10240
ZR = NPAD // NS
WR = 624
WREM = N - WR * NS

_MESH = dict(core_axis_name="c", subcore_axis_name="s", num_cores=NC,
             num_subcores=NS)


def _deg_body(dstb, zeros1, deg_out, idx_v, ones_v, acc, dsem0, dsem1):
    c = lax.axis_index("c")
    s = lax.axis_index("s")
    pltpu.sync_copy(zeros1, acc.at[pl.ds(s * ZR, ZR)])
    for k in range(CHUNK // 16):
        ones_v[pl.ds(k * 16, 16)] = jnp.ones((16,), jnp.float32)
    pltpu.sync_copy(dstb.at[c].at[s], idx_v)
    plsc.subcore_barrier()

    @pl.loop(0, NCH, step=2)
    def _(j):
        cp0 = pltpu.async_copy(ones_v, acc.at[idx_v.at[j]], dsem0, add=True)
        cp1 = pltpu.async_copy(ones_v, acc.at[idx_v.at[j + 1]], dsem1, add=True)
        cp0.wait()
        cp1.wait()

    plsc.subcore_barrier()
    pltpu.sync_copy(acc.at[pl.ds(s * ZR, ZR)], deg_out.at[c].at[pl.ds(s * ZR, ZR)])


def _agg_body(mpp, srcb, dstb, zeros2, agg_out, sidx, didx, r0, r1, r2, r3,
              acc, g0, g1, g2, g3, s0, s1, s2, s3):
    c = lax.axis_index("c")
    s = lax.axis_index("s")
    pltpu.sync_copy(zeros2, acc.at[pl.ds(s * ZR, ZR)])
    mpc = mpp.at[c]
    my_src = srcb.at[c].at[s]
    my_dst = dstb.at[c].at[s]
    plsc.subcore_barrier()
    rows = [r0, r1, r2, r3]
    gsem = [g0, g1, g2, g3]
    ssem = [s0, s1, s2, s3]

    def gwait(j, b):
        pltpu.make_async_copy(mpc.at[sidx.at[j]], rows[b], gsem[b]).wait()

    def ggo(j, b):
        pltpu.async_copy(mpc.at[sidx.at[j]], rows[b], gsem[b])

    def swait(j, b):
        pltpu.make_async_copy(rows[b], acc.at[didx.at[j]], ssem[b]).wait()

    def sgo(j, b):
        pltpu.async_copy(rows[b], acc.at[didx.at[j]], ssem[b], add=True)

    @pl.loop(0, NSB)
    def _(t):
        pltpu.sync_copy(my_src.at[pl.ds(t * SB, SB)], sidx)
        pltpu.sync_copy(my_dst.at[pl.ds(t * SB, SB)], didx)
        for b in range(4):
            ggo(b, b)
        gwait(0, 0); sgo(0, 0)
        gwait(1, 1); sgo(1, 1)
        gwait(2, 2); sgo(2, 2); swait(0, 0); ggo(4, 0)
        gwait(3, 3); sgo(3, 3); swait(1, 1); ggo(5, 1)

        @pl.loop(4, SB - 4, step=4)
        def _(j):
            for b in range(4):
                jj = j + b
                gwait(jj, b)
                sgo(jj, b)
                swait(jj - 2, (b + 2) % 4)
                ggo(jj + 2, (b + 2) % 4)

        gwait(SB - 4, 0); sgo(SB - 4, 0); swait(SB - 6, 2); ggo(SB - 2, 2)
        gwait(SB - 3, 1); sgo(SB - 3, 1); swait(SB - 5, 3); ggo(SB - 1, 3)
        gwait(SB - 2, 2); sgo(SB - 2, 2); swait(SB - 4, 0)
        gwait(SB - 1, 3); sgo(SB - 1, 3); swait(SB - 3, 1)
        swait(SB - 2, 2)
        swait(SB - 1, 3)

    plsc.subcore_barrier()
    pltpu.sync_copy(acc.at[pl.ds(s * WR, WR)], agg_out.at[c].at[pl.ds(s * WR, WR)])

    @pl.when(s == NS - 1)
    def _():
        pltpu.sync_copy(acc.at[pl.ds(WR * NS, WREM)],
                        agg_out.at[c].at[pl.ds(WR * NS, WREM)])


_deg_kernel = pl.kernel(
    _deg_body,
    out_type=jax.ShapeDtypeStruct((NC, NPAD), jnp.float32),
    mesh=plsc.VectorSubcoreMesh(**_MESH),
    scratch_types=[
        pltpu.VMEM((NCH, CHUNK), jnp.int32),
        pltpu.VMEM((CHUNK,), jnp.float32),
        pltpu.VMEM_SHARED((NPAD,), jnp.float32),
        pltpu.SemaphoreType.DMA,
        pltpu.SemaphoreType.DMA,
    ],
)

_agg_kernel = pl.kernel(
    _agg_body,
    out_type=jax.ShapeDtypeStruct((NC, N, D), jnp.float32),
    mesh=plsc.VectorSubcoreMesh(**_MESH),
    scratch_types=[
        pltpu.VMEM((SB, CHUNK), jnp.int32),
        pltpu.VMEM((SB, CHUNK), jnp.int32),
        pltpu.VMEM((CHUNK, D), jnp.float32),
        pltpu.VMEM((CHUNK, D), jnp.float32),
        pltpu.VMEM((CHUNK, D), jnp.float32),
        pltpu.VMEM((CHUNK, D), jnp.float32),
        pltpu.VMEM_SHARED((NPAD, D), jnp.float32),
    ] + [pltpu.SemaphoreType.DMA] * 8,
)


BLK = 1000
GRID = N // BLK


def _tc1_body(x_ref, w0_ref, w1_ref, dv_ref, mp_ref):
    xb = x_ref[...]
    dv = dv_ref[...]
    mp_ref[0] = dv[0] * jnp.dot(xb, w0_ref[...], preferred_element_type=jnp.float32)
    mp_ref[1] = dv[1] * jnp.dot(xb, w1_ref[...], preferred_element_type=jnp.float32)


def _tc2_body(agg_ref, mp_ref, dv_ref, b_ref, w0_ref, w1_ref, mp2_ref):
    agg = agg_ref[...]
    mp = mp_ref[...]
    dv = dv_ref[...]
    b = b_ref[...]
    h = dv[0] * (agg[0] + mp[0]) + b[0] + dv[1] * (agg[1] + mp[1]) + b[1]
    h = jnp.maximum(h, 0.0)
    mp2_ref[0] = dv[0] * jnp.dot(h, w0_ref[...], preferred_element_type=jnp.float32)
    mp2_ref[1] = dv[1] * jnp.dot(h, w1_ref[...], preferred_element_type=jnp.float32)


def _tc3_body(agg_ref, mp_ref, dv_ref, b_ref, out_ref):
    agg = agg_ref[...]
    mp = mp_ref[...]
    dv = dv_ref[...]
    b = b_ref[...]
    out_ref[...] = (dv[0] * (agg[0] + mp[0]) + b[0]
                    + dv[1] * (agg[1] + mp[1]) + b[1])


_spec_x = pl.BlockSpec((BLK, D), lambda i: (i, 0))
_spec_w = pl.BlockSpec((D, D), lambda i: (0, 0))
_spec_dv = pl.BlockSpec((NC, BLK, 1), lambda i: (0, i, 0))
_spec_mp = pl.BlockSpec((NC, BLK, D), lambda i: (0, i, 0))
_spec_b = pl.BlockSpec((NC, 1, D), lambda i: (0, 0, 0))
_spec_out = pl.BlockSpec((BLK, D), lambda i: (i, 0))

_tc1 = pl.pallas_call(
    _tc1_body,
    grid=(GRID,),
    in_specs=[_spec_x, _spec_w, _spec_w, _spec_dv],
    out_specs=_spec_mp,
    out_shape=jax.ShapeDtypeStruct((NC, N, D), jnp.float32),
)

_tc2 = pl.pallas_call(
    _tc2_body,
    grid=(GRID,),
    in_specs=[_spec_mp, _spec_mp, _spec_dv, _spec_b, _spec_w, _spec_w],
    out_specs=_spec_mp,
    out_shape=jax.ShapeDtypeStruct((NC, N, D), jnp.float32),
)

_tc3 = pl.pallas_call(
    _tc3_body,
    grid=(GRID,),
    in_specs=[_spec_mp, _spec_mp, _spec_dv, _spec_b],
    out_specs=_spec_out,
    out_shape=jax.ShapeDtypeStruct((N, D), jnp.float32),
)


def _prep_edges(ei):
    src = ei[0].reshape(NS, EPT)
    dst = ei[1].reshape(NS, EPT)
    pad = EPT_PAD - EPT
    src = jnp.pad(src, ((0, 0), (0, pad)))
    dst = jnp.pad(dst, ((0, 0), (0, pad)), constant_values=DUMP)
    return src.reshape(NS, NCH, CHUNK), dst.reshape(NS, NCH, CHUNK)


def kernel(x, edge_index_0, edge_index_1, W1_0, b1_0, W1_1, b1_1,
           W2_0, b2_0, W2_1, b2_1):
    s0, d0 = _prep_edges(edge_index_0)
    s1, d1 = _prep_edges(edge_index_1)
    srcb = jnp.stack([s0, s1])
    dstb = jnp.stack([d0, d1])
    zeros1 = jnp.zeros((ZR,), jnp.float32)
    zeros2 = jnp.zeros((ZR, D), jnp.float32)

    degc = _deg_kernel(dstb, zeros1)
    dinv = lax.rsqrt(degc[:, :N] + 1.0)
    dv = dinv[:, :, None]

    b1s = jnp.stack([b1_0, b1_1]).reshape(NC, 1, D)
    b2s = jnp.stack([b2_0, b2_1]).reshape(NC, 1, D)

    mp1 = _tc1(x, W1_0, W1_1, dv)
    agg1 = _agg_kernel(mp1, srcb, dstb, zeros2)
    mp2 = _tc2(agg1, mp1, dv, b1s, W2_0, W2_1)
    agg2 = _agg_kernel(mp2, srcb, dstb, zeros2)
    return _tc3(agg2, mp2, dv, b2s)
